# baseline (speedup 1.0000x reference)
import functools

import jax
import jax.numpy as jnp
from jax import lax
from jax.experimental import pallas as pl
from jax.experimental.pallas import tpu as pltpu
from jax.experimental.pallas import tpu_sc as plsc

BATCH = 16384
D_MODEL = 512
LANES = 16

NUM_CORES = 2
NUM_SUBCORES = 16
NUM_WORKERS = NUM_CORES * NUM_SUBCORES
B_PER_W = BATCH // NUM_WORKERS
CHUNK = 32
NCHUNKS = B_PER_W // CHUNK
NGBUF = 3
NXBUF = 4


def _body(x_hbm, idx_hbm, tbl_hbm, out_hbm,
          idx_v, xbuf, rbuf, gsem, xsem, ssem):
    wid = lax.axis_index("s") * NUM_CORES + lax.axis_index("c")
    base = wid * B_PER_W
    pltpu.sync_copy(idx_hbm.at[pl.ds(base, B_PER_W)], idx_v)

    def issue_gather(c):
        return pltpu.async_copy(
            tbl_hbm.at[idx_v.at[pl.ds(c * CHUNK, CHUNK)]],
            rbuf.at[c % NGBUF], gsem.at[c % NGBUF])

    def issue_xload(c):
        return pltpu.async_copy(
            x_hbm.at[pl.ds(base + c * CHUNK, CHUNK)],
            xbuf.at[c % NXBUF], xsem.at[c % NXBUF])

    gathers = {}
    xloads = {}
    stores = {}
    for c in range(NGBUF):
        gathers[c] = issue_gather(c)
    for c in range(NXBUF - 1):
        xloads[c] = issue_xload(c)

    for c in range(NCHUNKS):
        bg = c % NGBUF
        bx = c % NXBUF
        gathers.pop(c).wait()
        xloads.pop(c).wait()

        @plsc.parallel_loop(0, CHUNK, step=1, unroll=2 if c % 2 == 0 else 1)
        def add_row(i):
            for j in range(D_MODEL // LANES):
                sl = pl.ds(j * LANES, LANES)
                plsc.addupdate(xbuf.at[bx, i, sl], rbuf[bg, i, sl])

        stores[c] = pltpu.async_copy(
            xbuf.at[bx], out_hbm.at[pl.ds(base + c * CHUNK, CHUNK)],
            ssem.at[bx])
        if c + NGBUF < NCHUNKS:
            gathers[c + NGBUF] = issue_gather(c + NGBUF)
        if c + NXBUF - 1 < NCHUNKS:
            if c - 1 in stores:
                stores.pop(c - 1).wait()
            xloads[c + NXBUF - 1] = issue_xload(c + NXBUF - 1)
    for c in sorted(stores):
        stores.pop(c).wait()


def kernel(x, context_info, context_emb_weight):
    mesh = plsc.VectorSubcoreMesh(core_axis_name="c", subcore_axis_name="s")
    kfn = functools.partial(
        pl.kernel,
        mesh=mesh,
        out_type=jax.ShapeDtypeStruct((BATCH, D_MODEL), jnp.float32),
        scratch_types=[
            pltpu.VMEM((B_PER_W,), jnp.int32),
            pltpu.VMEM((NXBUF, CHUNK, D_MODEL), jnp.float32),
            pltpu.VMEM((NGBUF, CHUNK, D_MODEL), jnp.float32),
            pltpu.SemaphoreType.DMA((NGBUF,)),
            pltpu.SemaphoreType.DMA((NXBUF,)),
            pltpu.SemaphoreType.DMA((NXBUF,)),
        ],
    )(_body)
    return kfn(x, context_info.astype(jnp.int32), context_emb_weight)


# speedup vs baseline: 1.0738x; 1.0738x over previous
import functools

import jax
import jax.numpy as jnp
from jax import lax
from jax.experimental import pallas as pl
from jax.experimental.pallas import tpu as pltpu
from jax.experimental.pallas import tpu_sc as plsc

BATCH = 16384
D_MODEL = 512
LANES = 16

NUM_CORES = 2
NUM_SUBCORES = 16
NUM_WORKERS = NUM_CORES * NUM_SUBCORES
B_PER_W = BATCH // NUM_WORKERS
CHUNK = 32
NCHUNKS = B_PER_W // CHUNK
NGBUF = 3
NXBUF = 4


def _body(x_hbm, idx_hbm, tbl_hbm, out_hbm,
          idx_v, xbuf, rbuf, gsem, xsem, ssem):
    wid = lax.axis_index("c") * NUM_SUBCORES + lax.axis_index("s")
    base = wid * B_PER_W
    pltpu.sync_copy(idx_hbm.at[pl.ds(base, B_PER_W)], idx_v)

    def issue_gather(c):
        return pltpu.async_copy(
            tbl_hbm.at[idx_v.at[pl.ds(c * CHUNK, CHUNK)]],
            rbuf.at[c % NGBUF], gsem.at[c % NGBUF])

    def issue_xload(c):
        return pltpu.async_copy(
            x_hbm.at[pl.ds(base + c * CHUNK, CHUNK)],
            xbuf.at[c % NXBUF], xsem.at[c % NXBUF])

    gathers = {}
    xloads = {}
    stores = {}
    for c in range(NGBUF):
        gathers[c] = issue_gather(c)
    for c in range(NXBUF - 1):
        xloads[c] = issue_xload(c)

    for c in range(NCHUNKS):
        bg = c % NGBUF
        bx = c % NXBUF
        gathers.pop(c).wait()
        xloads.pop(c).wait()

        @plsc.parallel_loop(0, CHUNK, step=1)
        def add_row(i):
            for j in range(D_MODEL // LANES):
                sl = pl.ds(j * LANES, LANES)
                plsc.addupdate(xbuf.at[bx, i, sl], rbuf[bg, i, sl])

        stores[c] = pltpu.async_copy(
            xbuf.at[bx], out_hbm.at[pl.ds(base + c * CHUNK, CHUNK)],
            ssem.at[bx])
        if c + NGBUF < NCHUNKS:
            gathers[c + NGBUF] = issue_gather(c + NGBUF)
        if c + NXBUF - 1 < NCHUNKS:
            if c - 1 in stores:
                stores.pop(c - 1).wait()
            xloads[c + NXBUF - 1] = issue_xload(c + NXBUF - 1)
    for c in sorted(stores):
        stores.pop(c).wait()


def kernel(x, context_info, context_emb_weight):
    mesh = plsc.VectorSubcoreMesh(core_axis_name="c", subcore_axis_name="s")
    kfn = functools.partial(
        pl.kernel,
        mesh=mesh,
        out_type=jax.ShapeDtypeStruct((BATCH, D_MODEL), jnp.float32),
        scratch_types=[
            pltpu.VMEM((B_PER_W,), jnp.int32),
            pltpu.VMEM((NXBUF, CHUNK, D_MODEL), jnp.float32),
            pltpu.VMEM((NGBUF, CHUNK, D_MODEL), jnp.float32),
            pltpu.SemaphoreType.DMA((NGBUF,)),
            pltpu.SemaphoreType.DMA((NXBUF,)),
            pltpu.SemaphoreType.DMA((NXBUF,)),
        ],
    )(_body)
    return kfn(x, context_info.astype(jnp.int32), context_emb_weight)
